# Initial kernel scaffold; baseline (speedup 1.0000x reference)
import jax
import jax.numpy as jnp
from jax.experimental import pallas as pl


def kernel(features, embed_item, edge_index_clicks, edge_index_clicked_by, edge_index_follows, W0_clicks, b0_clicks, W0_clicked_by, b0_clicked_by, W0_follows, b0_follows, W1_clicks, b1_clicks, W1_clicked_by, b1_clicked_by, W1_follows, b1_follows, lin_W, lin_b):
    raise NotImplementedError("write your pallas kernel here")


# speedup vs baseline: 2.4388x; 2.4388x over previous
import functools

import jax
import jax.numpy as jnp
from jax import lax
from jax.experimental import pallas as pl
from jax.experimental.pallas import tpu as pltpu
from jax.experimental.pallas import tpu_sc as plsc

N = 50000
E = 400000
HID = 128
OUT = 16
NSUB = 16
NCORE = 2
BLK = 128
NBLK = E // BLK
CCH = 32
NCH = HID // CCH
ZR = 400
RPT = 3200

_MESH = plsc.VectorSubcoreMesh(core_axis_name="c", subcore_axis_name="s")


def _span(total, parts, i):
    base = total // parts
    rem = total % parts
    start = i * base + jnp.minimum(i, rem)
    cnt = base + jnp.where(i < rem, 1, 0).astype(jnp.int32)
    return start, cnt


def _fill(buf, rows, cols, value):
    v = jnp.full((16,), value, jnp.float32)

    def row(i, _):
        for c0 in range(0, cols, 16):
            buf[i, pl.ds(c0, 16)] = v
        return 0

    lax.fori_loop(0, rows, row, 0)


def _per_tile_rows(tid, fn):
    @pl.when(tid < NSUB - 1)
    def _():
        fn(tid * RPT, RPT // ZR)

    @pl.when(tid == NSUB - 1)
    def _():
        fn((NSUB - 1) * RPT, (N - (NSUB - 1) * RPT) // ZR)


def _zero_my_rows(acc_sp, zbuf, tid):
    def fn(row0, nch):
        for z in range(nch):
            pltpu.sync_copy(zbuf, acc_sp.at[pl.ds(row0 + z * ZR, ZR)])

    _per_tile_rows(tid, fn)


def _copy_my_rows(acc_sp, out_ref, tid):
    def fn(row0, nch):
        for z in range(nch):
            pltpu.sync_copy(acc_sp.at[pl.ds(row0 + z * ZR, ZR)],
                            out_ref.at[pl.ds(row0 + z * ZR, ZR)])

    _per_tile_rows(tid, fn)


def _accumulate(tab_h, src_h, dst_h, acc_sp, sbuf, dbuf, rbuf, start, cnt):
    def body(b, _):
        off = (start + b) * BLK
        pltpu.sync_copy(src_h.at[pl.ds(off, BLK)], sbuf.at[0])
        pltpu.sync_copy(dst_h.at[pl.ds(off, BLK)], dbuf.at[0])
        pltpu.sync_copy(tab_h.at[sbuf.at[0]], rbuf)
        pltpu.sync_copy(rbuf, acc_sp.at[dbuf.at[0]], add=True)
        return 0

    lax.fori_loop(0, cnt, body, 0)


def _seg128(t4, src, dst):
    @functools.partial(
        pl.kernel,
        out_type=jax.ShapeDtypeStruct((NCH, N, CCH), jnp.float32),
        mesh=_MESH,
        compiler_params=pltpu.CompilerParams(use_tc_tiling_on_sc=False),
        scratch_types=[
            pltpu.VMEM_SHARED((N, CCH), jnp.float32),
            pltpu.VMEM((ZR, CCH), jnp.float32),
            pltpu.VMEM((1, BLK), jnp.int32),
            pltpu.VMEM((1, BLK), jnp.int32),
            pltpu.VMEM((BLK, CCH), jnp.float32),
        ],
    )
    def k(t4_h, src_h, dst_h, out_h, acc_sp, zbuf, sbuf, dbuf, rbuf):
        cid = lax.axis_index("c")
        tid = lax.axis_index("s")
        _fill(zbuf, ZR, CCH, 0.0)
        start, cnt = _span(NBLK, NSUB, tid)

        def do_chunk(ci):
            _zero_my_rows(acc_sp, zbuf, tid)
            plsc.subcore_barrier()
            _accumulate(t4_h.at[ci], src_h, dst_h, acc_sp, sbuf, dbuf, rbuf,
                        start, cnt)
            plsc.subcore_barrier()
            _copy_my_rows(acc_sp, out_h.at[ci], tid)
            plsc.subcore_barrier()

        for j in range(2):
            @pl.when(cid == 0)
            def _():
                do_chunk(j)

            @pl.when(cid == 1)
            def _():
                do_chunk(2 + j)

    return k(t4, src, dst)


def _seg16(z0, z1, src0, dst0, src1, dst1):
    @functools.partial(
        pl.kernel,
        out_type=jax.ShapeDtypeStruct((4, N, OUT), jnp.float32),
        mesh=_MESH,
        compiler_params=pltpu.CompilerParams(use_tc_tiling_on_sc=False),
        scratch_types=[
            pltpu.VMEM_SHARED((N, OUT), jnp.float32),
            pltpu.VMEM((ZR, OUT), jnp.float32),
            pltpu.VMEM((1, BLK), jnp.int32),
            pltpu.VMEM((1, BLK), jnp.int32),
            pltpu.VMEM((BLK, OUT), jnp.float32),
        ],
    )
    def k(z0_h, z1_h, s0_h, d0_h, s1_h, d1_h, out_h, acc_sp, zbuf, sbuf, dbuf,
          rbuf):
        cid = lax.axis_index("c")
        tid = lax.axis_index("s")
        _fill(zbuf, ZR, OUT, 0.0)
        cstart, ccnt = _span(NBLK, NCORE, cid)
        tstart, tcnt = _span(ccnt, NSUB, tid)
        start = cstart + tstart

        for rel, (z_h, s_h, d_h) in enumerate(((z0_h, s0_h, d0_h),
                                               (z1_h, s1_h, d1_h))):
            _zero_my_rows(acc_sp, zbuf, tid)
            plsc.subcore_barrier()
            _accumulate(z_h, s_h, d_h, acc_sp, sbuf, dbuf, rbuf, start, tcnt)
            plsc.subcore_barrier()

            @pl.when(cid == 0)
            def _():
                _copy_my_rows(acc_sp, out_h.at[2 * rel], tid)

            @pl.when(cid == 1)
            def _():
                _copy_my_rows(acc_sp, out_h.at[2 * rel + 1], tid)

            plsc.subcore_barrier()

    return k(z0, z1, src0, dst0, src1, dst1)


def _counts(dst_c, dst_cb, dst_f):
    @functools.partial(
        pl.kernel,
        out_type=(jax.ShapeDtypeStruct((N, OUT), jnp.float32),) * 3,
        mesh=_MESH,
        compiler_params=pltpu.CompilerParams(use_tc_tiling_on_sc=False),
        scratch_types=[
            pltpu.VMEM_SHARED((N, OUT), jnp.float32),
            pltpu.VMEM_SHARED((N, OUT), jnp.float32),
            pltpu.VMEM((ZR, OUT), jnp.float32),
            pltpu.VMEM((BLK, OUT), jnp.float32),
            pltpu.VMEM((1, BLK), jnp.int32),
            pltpu.VMEM((ZR, OUT), jnp.float32),
        ],
    )
    def k(dc_h, dcb_h, df_h, qc_h, qcb_h, qf_h, acc0, acc1, zbuf, ones, dbuf,
          qbuf):
        cid = lax.axis_index("c")
        tid = lax.axis_index("s")
        _fill(zbuf, ZR, OUT, 0.0)
        _fill(ones, BLK, OUT, 1.0)
        start, cnt = _span(NBLK, NSUB, tid)

        def hist(d_h, acc):
            def body(b, _):
                off = (start + b) * BLK
                pltpu.sync_copy(d_h.at[pl.ds(off, BLK)], dbuf.at[0])
                pltpu.sync_copy(ones, acc.at[dbuf.at[0]], add=True)
                return 0

            lax.fori_loop(0, cnt, body, 0)

        def finalize(acc, q_h):
            def fn(row0, nch):
                for z in range(nch):
                    r0 = row0 + z * ZR
                    pltpu.sync_copy(acc.at[pl.ds(r0, ZR)], qbuf)

                    def row(i, _):
                        v = qbuf[i, pl.ds(0, 16)]
                        r = 1.0 / jnp.maximum(v, 1.0)
                        qbuf[i, pl.ds(0, 16)] = jnp.where(
                            v > 0.5, r, jnp.zeros((16,), jnp.float32))
                        return 0

                    lax.fori_loop(0, ZR, row, 0)
                    pltpu.sync_copy(qbuf, q_h.at[pl.ds(r0, ZR)])

            _per_tile_rows(tid, fn)

        @pl.when(cid == 0)
        def _():
            _zero_my_rows(acc0, zbuf, tid)
            _zero_my_rows(acc1, zbuf, tid)
            plsc.subcore_barrier()
            hist(dc_h, acc0)
            hist(dcb_h, acc1)
            plsc.subcore_barrier()
            finalize(acc0, qc_h)
            finalize(acc1, qcb_h)

        @pl.when(cid == 1)
        def _():
            _zero_my_rows(acc0, zbuf, tid)
            plsc.subcore_barrier()
            hist(df_h, acc0)
            plsc.subcore_barrier()
            finalize(acc0, qf_h)

    return k(dst_c, dst_cb, dst_f)


RB = 400
_GRID = N // RB


def _lrelu(x):
    return jnp.where(x >= 0, x, 0.01 * x)


def _chunk_mm(s, w):
    acc = jnp.dot(s[0], w[0:CCH, :], preferred_element_type=jnp.float32)
    for c in range(1, NCH):
        acc = acc + jnp.dot(s[c], w[c * CCH:(c + 1) * CCH, :],
                            preferred_element_type=jnp.float32)
    return acc


def _stage_b_body(sc_r, qc_r, scb_r, qcb_r, sf_r, qf_r, w0c_r, b0c_r, w0cb_r,
                  b0cb_r, w0f_r, b0f_r, w1cb_r, w1f_r, wlin_r, zi_r, zu_r):
    wlin = wlin_r[...]
    qc = qc_r[...][:, 0:1]
    gc = (qc > 0).astype(jnp.float32)
    item0 = qc * _chunk_mm(sc_r[...], w0c_r[...]) + gc * b0c_r[...]
    zi_r[...] = jnp.dot(_lrelu(item0),
                        jnp.dot(w1cb_r[...], wlin,
                                preferred_element_type=jnp.float32),
                        preferred_element_type=jnp.float32)
    qcb = qcb_r[...][:, 0:1]
    gcb = (qcb > 0).astype(jnp.float32)
    qf = qf_r[...][:, 0:1]
    gf = (qf > 0).astype(jnp.float32)
    user0 = (qcb * _chunk_mm(scb_r[...], w0cb_r[...]) + gcb * b0cb_r[...]
             + qf * _chunk_mm(sf_r[...], w0f_r[...]) + gf * b0f_r[...])
    zu_r[...] = jnp.dot(_lrelu(user0),
                        jnp.dot(w1f_r[...], wlin,
                                preferred_element_type=jnp.float32),
                        preferred_element_type=jnp.float32)


def _stage_b(sc, qc, scb, qcb, sf, qf, w0c, b0c, w0cb, b0cb, w0f, b0f, w1cb,
             w1f, wlin):
    s_spec = pl.BlockSpec((NCH, RB, CCH), lambda r: (0, r, 0))
    q_spec = pl.BlockSpec((RB, OUT), lambda r: (r, 0))
    w_spec = pl.BlockSpec((HID, HID), lambda r: (0, 0))
    b_spec = pl.BlockSpec((1, HID), lambda r: (0, 0))
    return pl.pallas_call(
        _stage_b_body,
        grid=(_GRID,),
        in_specs=[s_spec, q_spec, s_spec, q_spec, s_spec, q_spec,
                  w_spec, b_spec, w_spec, b_spec, w_spec, b_spec,
                  w_spec, w_spec, pl.BlockSpec((HID, OUT), lambda r: (0, 0))],
        out_specs=[q_spec, q_spec],
        out_shape=[jax.ShapeDtypeStruct((N, OUT), jnp.float32)] * 2,
    )(sc, qc, scb, qcb, sf, qf, w0c, b0c, w0cb, b0cb, w0f, b0f, w1cb, w1f,
      wlin)


def _stage_d_body(p_r, qcb_r, qf_r, b1cb_r, b1f_r, wlin_r, linb_r, out_r):
    p = p_r[...]
    qcb = qcb_r[...][:, 0:1]
    gcb = (qcb > 0).astype(jnp.float32)
    qf = qf_r[...][:, 0:1]
    gf = (qf > 0).astype(jnp.float32)
    wlin = wlin_r[...]
    bias = (gcb * jnp.dot(b1cb_r[...], wlin, preferred_element_type=jnp.float32)
            + gf * jnp.dot(b1f_r[...], wlin, preferred_element_type=jnp.float32)
            + linb_r[...])
    out_r[...] = qcb * (p[0] + p[1]) + qf * (p[2] + p[3]) + bias


def _stage_d(p, qcb, qf, b1cb, b1f, wlin, linb):
    q_spec = pl.BlockSpec((RB, OUT), lambda r: (r, 0))
    return pl.pallas_call(
        _stage_d_body,
        grid=(_GRID,),
        in_specs=[pl.BlockSpec((4, RB, OUT), lambda r: (0, r, 0)), q_spec,
                  q_spec, pl.BlockSpec((1, HID), lambda r: (0, 0)),
                  pl.BlockSpec((1, HID), lambda r: (0, 0)),
                  pl.BlockSpec((HID, OUT), lambda r: (0, 0)),
                  pl.BlockSpec((1, OUT), lambda r: (0, 0))],
        out_specs=q_spec,
        out_shape=jax.ShapeDtypeStruct((N, OUT), jnp.float32),
    )(p, qcb, qf, b1cb, b1f, wlin, linb)


def kernel(features, embed_item, edge_index_clicks, edge_index_clicked_by,
           edge_index_follows, W0_clicks, b0_clicks, W0_clicked_by,
           b0_clicked_by, W0_follows, b0_follows, W1_clicks, b1_clicks,
           W1_clicked_by, b1_clicked_by, W1_follows, b1_follows, lin_W,
           lin_b):
    i32 = jnp.int32
    sc_, dc_ = (edge_index_clicks[0].astype(i32),
                edge_index_clicks[1].astype(i32))
    scb, dcb = (edge_index_clicked_by[0].astype(i32),
                edge_index_clicked_by[1].astype(i32))
    sf_, df_ = (edge_index_follows[0].astype(i32),
                edge_index_follows[1].astype(i32))

    f4 = features.reshape(N, NCH, CCH).transpose(1, 0, 2)
    e4 = embed_item.reshape(N, NCH, CCH).transpose(1, 0, 2)

    qc, qcb, qf = _counts(dc_, dcb, df_)
    s_clicks = _seg128(f4, sc_, dc_)
    s_cb = _seg128(e4, scb, dcb)
    s_f = _seg128(f4, sf_, df_)

    zi, zu = _stage_b(s_clicks, qc, s_cb, qcb, s_f, qf,
                      W0_clicks, b0_clicks.reshape(1, HID),
                      W0_clicked_by, b0_clicked_by.reshape(1, HID),
                      W0_follows, b0_follows.reshape(1, HID),
                      W1_clicked_by, W1_follows, lin_W)

    p = _seg16(zi, zu, scb, dcb, sf_, df_)

    return _stage_d(p, qcb, qf, b1_clicked_by.reshape(1, HID),
                    b1_follows.reshape(1, HID), lin_W,
                    lin_b.reshape(1, OUT))


# speedup vs baseline: 4.7220x; 1.9362x over previous
import functools

import jax
import jax.numpy as jnp
from jax import lax
from jax.experimental import pallas as pl
from jax.experimental.pallas import tpu as pltpu
from jax.experimental.pallas import tpu_sc as plsc

N = 50000
E = 400000
HID = 128
OUT = 16
NSUB = 16
NCORE = 2
BLK = 128
NBLK = E // BLK
CCH = 32
NCH = HID // CCH
ZR = 200
RPT = 3200

_MESH = plsc.VectorSubcoreMesh(core_axis_name="c", subcore_axis_name="s")


def _span(total, parts, i):
    base = total // parts
    rem = total % parts
    start = i * base + jnp.minimum(i, rem)
    cnt = base + jnp.where(i < rem, 1, 0).astype(jnp.int32)
    return start, cnt


def _fill(buf, rows, cols, value):
    v = jnp.full((16,), value, jnp.float32)

    def row(i, _):
        for c0 in range(0, cols, 16):
            buf[i, pl.ds(c0, 16)] = v
        return 0

    lax.fori_loop(0, rows, row, 0)


def _per_tile_rows(tid, fn):
    @pl.when(tid < NSUB - 1)
    def _():
        fn(tid * RPT, RPT // ZR)

    @pl.when(tid == NSUB - 1)
    def _():
        fn((NSUB - 1) * RPT, (N - (NSUB - 1) * RPT) // ZR)


def _zero_my_rows(acc_sp, zbuf, tid):
    def fn(row0, nch):
        for z in range(nch):
            pltpu.sync_copy(zbuf, acc_sp.at[pl.ds(row0 + z * ZR, ZR)])

    _per_tile_rows(tid, fn)


def _copy_my_rows(acc_sp, out_ref, tid):
    def fn(row0, nch):
        for z in range(nch):
            pltpu.sync_copy(acc_sp.at[pl.ds(row0 + z * ZR, ZR)],
                            out_ref.at[pl.ds(row0 + z * ZR, ZR)])

    _per_tile_rows(tid, fn)


G128 = 4
G16 = 16


def _accumulate(gg, tab_h, src2_h, dst2_h, acc_sp, sgbuf, dgbuf, rbuf, gsem,
                ssem, start, cnt):
    def group(base, nb):
        pltpu.sync_copy(src2_h.at[pl.ds(base, nb)], sgbuf.at[pl.ds(0, nb)])
        pltpu.sync_copy(dst2_h.at[pl.ds(base, nb)], dgbuf.at[pl.ds(0, nb)])
        gds = [pltpu.async_copy(tab_h.at[sgbuf.at[j]],
                                rbuf.at[pl.ds(j * BLK, BLK)], gsem)
               for j in range(nb)]
        for d in gds:
            d.wait()
        sds = [pltpu.async_copy(rbuf.at[pl.ds(j * BLK, BLK)],
                                acc_sp.at[dgbuf.at[j]], ssem, add=True)
               for j in range(nb)]
        for d in sds:
            d.wait()

    ngrp = cnt // gg

    def body(g, _):
        group(start + g * gg, gg)
        return 0

    lax.fori_loop(0, ngrp, body, 0)

    def tail(t, _):
        group(start + ngrp * gg + t, 1)
        return 0

    lax.fori_loop(0, cnt - ngrp * gg, tail, 0)


def _seg128(t4, src, dst):
    @functools.partial(
        pl.kernel,
        out_type=jax.ShapeDtypeStruct((NCH, N, CCH), jnp.float32),
        mesh=_MESH,
        compiler_params=pltpu.CompilerParams(use_tc_tiling_on_sc=False),
        scratch_types=[
            pltpu.VMEM_SHARED((N, CCH), jnp.float32),
            pltpu.VMEM((ZR, CCH), jnp.float32),
            pltpu.VMEM((G128, BLK), jnp.int32),
            pltpu.VMEM((G128, BLK), jnp.int32),
            pltpu.VMEM((G128 * BLK, CCH), jnp.float32),
            pltpu.SemaphoreType.DMA,
            pltpu.SemaphoreType.DMA,
        ],
    )
    def k(t4_h, src_h, dst_h, out_h, acc_sp, zbuf, sbuf, dbuf, rbuf, gsem,
          ssem):
        cid = lax.axis_index("c")
        tid = lax.axis_index("s")
        _fill(zbuf, ZR, CCH, 0.0)
        start, cnt = _span(NBLK, NSUB, tid)

        def do_chunk(ci):
            _zero_my_rows(acc_sp, zbuf, tid)
            plsc.subcore_barrier()
            _accumulate(G128, t4_h.at[ci], src_h, dst_h, acc_sp, sbuf, dbuf,
                        rbuf, gsem, ssem, start, cnt)
            plsc.subcore_barrier()
            _copy_my_rows(acc_sp, out_h.at[ci], tid)
            plsc.subcore_barrier()

        for j in range(2):
            @pl.when(cid == 0)
            def _():
                do_chunk(j)

            @pl.when(cid == 1)
            def _():
                do_chunk(2 + j)

    return k(t4, src, dst)


def _seg16(z0, z1, src0, dst0, src1, dst1):
    @functools.partial(
        pl.kernel,
        out_type=jax.ShapeDtypeStruct((4, N, OUT), jnp.float32),
        mesh=_MESH,
        compiler_params=pltpu.CompilerParams(use_tc_tiling_on_sc=False),
        scratch_types=[
            pltpu.VMEM_SHARED((N, OUT), jnp.float32),
            pltpu.VMEM((ZR, OUT), jnp.float32),
            pltpu.VMEM((G16, BLK), jnp.int32),
            pltpu.VMEM((G16, BLK), jnp.int32),
            pltpu.VMEM((G16 * BLK, OUT), jnp.float32),
            pltpu.SemaphoreType.DMA,
            pltpu.SemaphoreType.DMA,
        ],
    )
    def k(z0_h, z1_h, s0_h, d0_h, s1_h, d1_h, out_h, acc_sp, zbuf, sbuf, dbuf,
          rbuf, gsem, ssem):
        cid = lax.axis_index("c")
        tid = lax.axis_index("s")
        _fill(zbuf, ZR, OUT, 0.0)
        cstart, ccnt = _span(NBLK, NCORE, cid)
        tstart, tcnt = _span(ccnt, NSUB, tid)
        start = cstart + tstart

        for rel, (z_h, s_h, d_h) in enumerate(((z0_h, s0_h, d0_h),
                                               (z1_h, s1_h, d1_h))):
            _zero_my_rows(acc_sp, zbuf, tid)
            plsc.subcore_barrier()
            _accumulate(G16, z_h, s_h, d_h, acc_sp, sbuf, dbuf, rbuf, gsem,
                        ssem, start, tcnt)
            plsc.subcore_barrier()

            @pl.when(cid == 0)
            def _():
                _copy_my_rows(acc_sp, out_h.at[2 * rel], tid)

            @pl.when(cid == 1)
            def _():
                _copy_my_rows(acc_sp, out_h.at[2 * rel + 1], tid)

            plsc.subcore_barrier()

    return k(z0, z1, src0, dst0, src1, dst1)


def _counts(dst_c, dst_cb, dst_f):
    @functools.partial(
        pl.kernel,
        out_type=(jax.ShapeDtypeStruct((N, OUT), jnp.float32),) * 3,
        mesh=_MESH,
        compiler_params=pltpu.CompilerParams(use_tc_tiling_on_sc=False),
        scratch_types=[
            pltpu.VMEM_SHARED((N, OUT), jnp.float32),
            pltpu.VMEM_SHARED((N, OUT), jnp.float32),
            pltpu.VMEM((ZR, OUT), jnp.float32),
            pltpu.VMEM((BLK, OUT), jnp.float32),
            pltpu.VMEM((G16, BLK), jnp.int32),
            pltpu.VMEM((ZR, OUT), jnp.float32),
            pltpu.SemaphoreType.DMA,
        ],
    )
    def k(dc_h, dcb_h, df_h, qc_h, qcb_h, qf_h, acc0, acc1, zbuf, ones, dbuf,
          qbuf, ssem):
        cid = lax.axis_index("c")
        tid = lax.axis_index("s")
        _fill(zbuf, ZR, OUT, 0.0)
        _fill(ones, BLK, OUT, 1.0)
        start, cnt = _span(NBLK, NSUB, tid)

        def hist(d_h, acc):
            def group(base, nb):
                pltpu.sync_copy(d_h.at[pl.ds(base, nb)],
                                dbuf.at[pl.ds(0, nb)])
                sds = [pltpu.async_copy(ones, acc.at[dbuf.at[j]], ssem,
                                        add=True) for j in range(nb)]
                for d in sds:
                    d.wait()

            ngrp = cnt // G16

            def body(g, _):
                group(start + g * G16, G16)
                return 0

            lax.fori_loop(0, ngrp, body, 0)

            def tailb(t, _):
                group(start + ngrp * G16 + t, 1)
                return 0

            lax.fori_loop(0, cnt - ngrp * G16, tailb, 0)

        def finalize(acc, q_h):
            def fn(row0, nch):
                for z in range(nch):
                    r0 = row0 + z * ZR
                    pltpu.sync_copy(acc.at[pl.ds(r0, ZR)], qbuf)

                    def row(i, _):
                        v = qbuf[i, pl.ds(0, 16)]
                        r = 1.0 / jnp.maximum(v, 1.0)
                        qbuf[i, pl.ds(0, 16)] = jnp.where(
                            v > 0.5, r, jnp.zeros((16,), jnp.float32))
                        return 0

                    lax.fori_loop(0, ZR, row, 0)
                    pltpu.sync_copy(qbuf, q_h.at[pl.ds(r0, ZR)])

            _per_tile_rows(tid, fn)

        @pl.when(cid == 0)
        def _():
            _zero_my_rows(acc0, zbuf, tid)
            _zero_my_rows(acc1, zbuf, tid)
            plsc.subcore_barrier()
            hist(dc_h, acc0)
            hist(dcb_h, acc1)
            plsc.subcore_barrier()
            finalize(acc0, qc_h)
            finalize(acc1, qcb_h)

        @pl.when(cid == 1)
        def _():
            _zero_my_rows(acc0, zbuf, tid)
            plsc.subcore_barrier()
            hist(df_h, acc0)
            plsc.subcore_barrier()
            finalize(acc0, qf_h)

    return k(dst_c, dst_cb, dst_f)


RB = 400
_GRID = N // RB


def _lrelu(x):
    return jnp.where(x >= 0, x, 0.01 * x)


def _chunk_mm(s, w):
    acc = jnp.dot(s[0], w[0:CCH, :], preferred_element_type=jnp.float32)
    for c in range(1, NCH):
        acc = acc + jnp.dot(s[c], w[c * CCH:(c + 1) * CCH, :],
                            preferred_element_type=jnp.float32)
    return acc


def _stage_b_body(sc_r, qc_r, scb_r, qcb_r, sf_r, qf_r, w0c_r, b0c_r, w0cb_r,
                  b0cb_r, w0f_r, b0f_r, w1cb_r, w1f_r, wlin_r, zi_r, zu_r):
    wlin = wlin_r[...]
    qc = qc_r[...][:, 0:1]
    gc = (qc > 0).astype(jnp.float32)
    item0 = qc * _chunk_mm(sc_r[...], w0c_r[...]) + gc * b0c_r[...]
    zi_r[...] = jnp.dot(_lrelu(item0),
                        jnp.dot(w1cb_r[...], wlin,
                                preferred_element_type=jnp.float32),
                        preferred_element_type=jnp.float32)
    qcb = qcb_r[...][:, 0:1]
    gcb = (qcb > 0).astype(jnp.float32)
    qf = qf_r[...][:, 0:1]
    gf = (qf > 0).astype(jnp.float32)
    user0 = (qcb * _chunk_mm(scb_r[...], w0cb_r[...]) + gcb * b0cb_r[...]
             + qf * _chunk_mm(sf_r[...], w0f_r[...]) + gf * b0f_r[...])
    zu_r[...] = jnp.dot(_lrelu(user0),
                        jnp.dot(w1f_r[...], wlin,
                                preferred_element_type=jnp.float32),
                        preferred_element_type=jnp.float32)


def _stage_b(sc, qc, scb, qcb, sf, qf, w0c, b0c, w0cb, b0cb, w0f, b0f, w1cb,
             w1f, wlin):
    s_spec = pl.BlockSpec((NCH, RB, CCH), lambda r: (0, r, 0))
    q_spec = pl.BlockSpec((RB, OUT), lambda r: (r, 0))
    w_spec = pl.BlockSpec((HID, HID), lambda r: (0, 0))
    b_spec = pl.BlockSpec((1, HID), lambda r: (0, 0))
    return pl.pallas_call(
        _stage_b_body,
        grid=(_GRID,),
        in_specs=[s_spec, q_spec, s_spec, q_spec, s_spec, q_spec,
                  w_spec, b_spec, w_spec, b_spec, w_spec, b_spec,
                  w_spec, w_spec, pl.BlockSpec((HID, OUT), lambda r: (0, 0))],
        out_specs=[q_spec, q_spec],
        out_shape=[jax.ShapeDtypeStruct((N, OUT), jnp.float32)] * 2,
    )(sc, qc, scb, qcb, sf, qf, w0c, b0c, w0cb, b0cb, w0f, b0f, w1cb, w1f,
      wlin)


def _stage_d_body(p_r, qcb_r, qf_r, b1cb_r, b1f_r, wlin_r, linb_r, out_r):
    p = p_r[...]
    qcb = qcb_r[...][:, 0:1]
    gcb = (qcb > 0).astype(jnp.float32)
    qf = qf_r[...][:, 0:1]
    gf = (qf > 0).astype(jnp.float32)
    wlin = wlin_r[...]
    bias = (gcb * jnp.dot(b1cb_r[...], wlin, preferred_element_type=jnp.float32)
            + gf * jnp.dot(b1f_r[...], wlin, preferred_element_type=jnp.float32)
            + linb_r[...])
    out_r[...] = qcb * (p[0] + p[1]) + qf * (p[2] + p[3]) + bias


def _stage_d(p, qcb, qf, b1cb, b1f, wlin, linb):
    q_spec = pl.BlockSpec((RB, OUT), lambda r: (r, 0))
    return pl.pallas_call(
        _stage_d_body,
        grid=(_GRID,),
        in_specs=[pl.BlockSpec((4, RB, OUT), lambda r: (0, r, 0)), q_spec,
                  q_spec, pl.BlockSpec((1, HID), lambda r: (0, 0)),
                  pl.BlockSpec((1, HID), lambda r: (0, 0)),
                  pl.BlockSpec((HID, OUT), lambda r: (0, 0)),
                  pl.BlockSpec((1, OUT), lambda r: (0, 0))],
        out_specs=q_spec,
        out_shape=jax.ShapeDtypeStruct((N, OUT), jnp.float32),
    )(p, qcb, qf, b1cb, b1f, wlin, linb)


def kernel(features, embed_item, edge_index_clicks, edge_index_clicked_by,
           edge_index_follows, W0_clicks, b0_clicks, W0_clicked_by,
           b0_clicked_by, W0_follows, b0_follows, W1_clicks, b1_clicks,
           W1_clicked_by, b1_clicked_by, W1_follows, b1_follows, lin_W,
           lin_b):
    i32 = jnp.int32
    r2 = lambda x: x.astype(i32).reshape(NBLK, BLK)
    sc_, dc_ = r2(edge_index_clicks[0]), r2(edge_index_clicks[1])
    scb, dcb = r2(edge_index_clicked_by[0]), r2(edge_index_clicked_by[1])
    sf_, df_ = r2(edge_index_follows[0]), r2(edge_index_follows[1])

    f4 = features.reshape(N, NCH, CCH).transpose(1, 0, 2)
    e4 = embed_item.reshape(N, NCH, CCH).transpose(1, 0, 2)

    qc, qcb, qf = _counts(dc_, dcb, df_)
    s_clicks = _seg128(f4, sc_, dc_)
    s_cb = _seg128(e4, scb, dcb)
    s_f = _seg128(f4, sf_, df_)

    zi, zu = _stage_b(s_clicks, qc, s_cb, qcb, s_f, qf,
                      W0_clicks, b0_clicks.reshape(1, HID),
                      W0_clicked_by, b0_clicked_by.reshape(1, HID),
                      W0_follows, b0_follows.reshape(1, HID),
                      W1_clicked_by, W1_follows, lin_W)

    p = _seg16(zi, zu, scb, dcb, sf_, df_)

    return _stage_d(p, qcb, qf, b1_clicked_by.reshape(1, HID),
                    b1_follows.reshape(1, HID), lin_W,
                    lin_b.reshape(1, OUT))
